# baseline (speedup 1.0000x reference)
import functools

import jax
import jax.numpy as jnp
from jax import lax
from jax.experimental import pallas as pl
from jax.experimental.pallas import tpu as pltpu
from jax.experimental.pallas import tpu_sc as plsc

B = 4096
E = 64
NC, NS, L = 2, 16, 16
NW = NC * NS
UPW = B // NW
LP = 80
NG = LP // L
FETCH = 88
RATW = 96

_F32 = jnp.float32
_I32 = jnp.int32


def _vsort(x):
  return jnp.sort(x)


def _rev(x):
  return lax.rev(x, (0,))


def _merge2(a, b):
  br = _rev(b)
  return _vsort(jnp.minimum(a, br)), _vsort(jnp.maximum(a, br))


def _sort48(v):
  s = [_vsort(x) for x in v]
  a0, a1 = _merge2(s[0], s[1])
  b0, b1 = _merge2(s[2], s[3])
  r0, r1 = _rev(b1), _rev(b0)
  l0, l1 = jnp.minimum(a0, r0), jnp.minimum(a1, r1)
  h0, h1 = jnp.maximum(a0, r0), jnp.maximum(a1, r1)
  c0 = _vsort(jnp.minimum(l0, l1))
  c1 = _vsort(jnp.maximum(l0, l1))
  c2 = _vsort(jnp.minimum(h0, h1))
  rb = _rev(s[4])
  p0, p1 = jnp.minimum(c0, c2), jnp.minimum(c1, rb)
  q0, q1 = jnp.maximum(c0, c2), jnp.maximum(c1, rb)
  return (_vsort(jnp.minimum(p0, p1)), _vsort(jnp.maximum(p0, p1)),
          _vsort(jnp.minimum(q0, q1)))


def _sc_body(hist_hbm, rat_hbm, cu_hbm, film_hbm, table_hbm,
             ue_hbm, fe_hbm,
             idx_a, rat_a, rows_a, idx_b, rat_b, rows_b,
             chmaj, sorted_v, cu_v, film_v, ferows_v, ue_v, sem):
  wid = lax.axis_index("s") * NC + lax.axis_index("c")
  u0 = wid * UPW

  pltpu.sync_copy(cu_hbm.at[pl.ds(u0, UPW + 16)], cu_v)
  pltpu.sync_copy(film_hbm.at[pl.ds(u0, UPW)], film_v)
  pltpu.async_copy(table_hbm.at[film_v], ferows_v, sem).wait()
  pltpu.sync_copy(ferows_v, fe_hbm.at[pl.ds(u0, UPW)])

  iota = lax.iota(_I32, L)
  inf16 = jnp.full((L,), jnp.inf, _F32)
  bufs = ((idx_a, rat_a, rows_a), (idx_b, rat_b, rows_b))

  def start_fetch(u, buf):
    bi, br, brw = buf
    cu_u = cu_v[pl.ds(u, L)][0]
    base = (cu_u // 8) * 8
    pltpu.sync_copy(hist_hbm.at[pl.ds(base, FETCH)], bi)
    pltpu.sync_copy(rat_hbm.at[pl.ds(base, FETCH)], br.at[0, pl.ds(0, FETCH)])
    pltpu.async_copy(table_hbm.at[bi], brw, sem)

  def process(u, cur, nxt):
    _, rat_v, rows_v = cur
    pltpu.make_async_copy(table_hbm.at[pl.ds(0, FETCH)], rows_v, sem).wait()

    @pl.when(u + 1 < UPW)
    def _prefetch():
      start_fetch(u + 1, nxt)

    cu_pair = cu_v[pl.ds(u, L)]
    cu_u = cu_pair[0]
    ln = cu_pair[1] - cu_u
    off = cu_u - (cu_u // 8) * 8

    chs = [iota + (g * L) for g in range(4)]
    zero16 = jnp.zeros((L,), _F32)
    ninf16 = jnp.full((L,), -jnp.inf, _F32)

    def t_main(t, carry):
      acc, mx = carry
      r = rat_v[0, pl.ds(off + t, L)][0]
      tvec = jnp.full((L,), t, _I32)
      accn, mxn = [], []
      for g in range(4):
        w = rows_v[off + t, pl.ds(g * L, L)] * r
        plsc.store_scatter(chmaj, [chs[g], tvec], w)
        accn.append(acc[g] + w)
        mxn.append(jnp.maximum(mx[g], w))
      return tuple(accn), tuple(mxn)

    acc, mx = lax.fori_loop(0, ln, t_main, ((zero16,) * 4, (ninf16,) * 4))

    def t_pad(t, _):
      tvec = jnp.full((L,), t, _I32)
      for g in range(4):
        plsc.store_scatter(chmaj, [chs[g], tvec], inf16)
      return 0

    lax.fori_loop(ln, LP, t_pad, 0)

    def chan_body(c, _):
      v = [chmaj[c, pl.ds(g * L, L)] for g in range(NG)]
      srt = _sort48(v)
      for j in range(3):
        sorted_v[c, pl.ds(j * L, L)] = srt[j]
      return 0

    lax.fori_loop(0, E, chan_body, 0, unroll=4)

    ln_vec = jnp.full((L,), ln, _I32).astype(_F32)
    k_med = jnp.full((L,), (ln - 1) // 2, _I32)
    k_min = jnp.zeros((L,), _I32)
    for g in range(4):
      ue_v[u, pl.ds(g * L, L)] = plsc.load_gather(sorted_v, [chs[g], k_min])
      ue_v[u, pl.ds(E + g * L, L)] = mx[g]
      ue_v[u, pl.ds(2 * E + g * L, L)] = acc[g] / ln_vec
      ue_v[u, pl.ds(3 * E + g * L, L)] = plsc.load_gather(
          sorted_v, [chs[g], k_med])

  start_fetch(0, bufs[0])

  def pair_body(v, _):
    process(2 * v, bufs[0], bufs[1])
    process(2 * v + 1, bufs[1], bufs[0])
    return 0

  lax.fori_loop(0, UPW // 2, pair_body, 0)
  pltpu.sync_copy(ue_v, ue_hbm.at[pl.ds(u0, UPW)])


def _sc_stats(hist_idx, ratings, cu, film, table):
  mesh = plsc.VectorSubcoreMesh(core_axis_name="c", subcore_axis_name="s")
  fn = functools.partial(
      pl.kernel,
      mesh=mesh,
      compiler_params=pltpu.CompilerParams(
          needs_layout_passes=False, use_tc_tiling_on_sc=False),
      out_type=[
          jax.ShapeDtypeStruct((B, 4 * E), _F32),
          jax.ShapeDtypeStruct((B, E), _F32),
      ],
      scratch_types=[
          pltpu.VMEM((FETCH,), _I32),
          pltpu.VMEM((1, RATW), _F32),
          pltpu.VMEM((FETCH, E), _F32),
          pltpu.VMEM((FETCH,), _I32),
          pltpu.VMEM((1, RATW), _F32),
          pltpu.VMEM((FETCH, E), _F32),
          pltpu.VMEM((E, 81), _F32),
          pltpu.VMEM((E, 49), _F32),
          pltpu.VMEM((UPW + 16,), _I32),
          pltpu.VMEM((UPW,), _I32),
          pltpu.VMEM((UPW, E), _F32),
          pltpu.VMEM((UPW, 4 * E), _F32),
          pltpu.SemaphoreType.DMA,
      ],
  )(_sc_body)
  return fn(hist_idx, ratings, cu, film, table)


def _mlp_body(ue_ref, fe_ref, w1_ref, b1_ref, w2_ref, b2_ref, w3_ref, b3_ref,
              out_ref):
  ue = ue_ref[...]
  fe = fe_ref[...]
  ss = jnp.sum(ue * ue, axis=1, keepdims=True)
  uen = ue * lax.rsqrt(ss)
  x = jnp.concatenate([uen, fe], axis=1)
  h = jnp.dot(x, w1_ref[...], preferred_element_type=_F32,
              precision=lax.Precision.HIGHEST) + b1_ref[...][None, :]
  h = jnp.maximum(h, 0.0)
  h = jnp.dot(h, w2_ref[...], preferred_element_type=_F32,
              precision=lax.Precision.HIGHEST) + b2_ref[...][None, :]
  h = jnp.maximum(h, 0.0)
  o = jnp.sum(h * w3_ref[...][None, :], axis=1) + b3_ref[...]
  out_ref[...] = 1.0 / (1.0 + jnp.exp(-o))


def _mlp(ue, fe, W1, b1, W2, b2, W3, b3):
  UB = 512
  grid = (B // UB,)
  return pl.pallas_call(
      _mlp_body,
      grid=grid,
      in_specs=[
          pl.BlockSpec((UB, 4 * E), lambda i: (i, 0)),
          pl.BlockSpec((UB, E), lambda i: (i, 0)),
          pl.BlockSpec((5 * E, 2 * E), lambda i: (0, 0)),
          pl.BlockSpec((2 * E,), lambda i: (0,)),
          pl.BlockSpec((2 * E, E), lambda i: (0, 0)),
          pl.BlockSpec((E,), lambda i: (0,)),
          pl.BlockSpec((E,), lambda i: (0,)),
          pl.BlockSpec((1,), lambda i: (0,)),
      ],
      out_specs=pl.BlockSpec((UB,), lambda i: (i,)),
      out_shape=jax.ShapeDtypeStruct((B,), _F32),
  )(ue, fe, W1, b1, W2, b2, W3[:, 0], b3)


def kernel(hist_film_indices, hist_ratings, seg_ids, cu_seqlens, lengths,
           film_indices, table, W1, b1, W2, b2, W3, b3):
  del seg_ids, lengths
  total = hist_film_indices.shape[0]
  hist_p = jnp.concatenate(
      [hist_film_indices.astype(_I32), jnp.zeros((FETCH,), _I32)])
  rat_p = jnp.concatenate(
      [hist_ratings.astype(_F32), jnp.zeros((FETCH,), _F32)])
  cu_p = jnp.concatenate(
      [cu_seqlens.astype(_I32), jnp.full((16,), total, _I32)])
  ue, fe = _sc_stats(hist_p, rat_p, cu_p, film_indices.astype(_I32), table)
  return _mlp(ue, fe, W1, b1, W2, b2, W3, b3)


# speedup vs baseline: 1.1662x; 1.1662x over previous
import functools

import jax
import jax.numpy as jnp
from jax import lax
from jax.experimental import pallas as pl
from jax.experimental.pallas import tpu as pltpu
from jax.experimental.pallas import tpu_sc as plsc

B = 4096
E = 64
NC, NS, L = 2, 16, 16
NW = NC * NS
UPW = B // NW
LP = 80
NG = LP // L
FETCH = 88
RATW = 96

_F32 = jnp.float32
_I32 = jnp.int32


def _vsort(x):
  return jnp.sort(x)


def _rev(x):
  return lax.rev(x, (0,))


def _merge2(a, b):
  br = _rev(b)
  return _vsort(jnp.minimum(a, br)), _vsort(jnp.maximum(a, br))


def _sort_low(v):
  m = len(v)
  s = [_vsort(x) for x in v]
  if m == 2:
    return list(_merge2(s[0], s[1]))
  if m == 3:
    a0, a1 = _merge2(s[0], s[1])
    l1 = jnp.minimum(a1, _rev(s[2]))
    return [_vsort(jnp.minimum(a0, l1)), _vsort(jnp.maximum(a0, l1))]
  a0, a1 = _merge2(s[0], s[1])
  b0, b1 = _merge2(s[2], s[3])
  r0, r1 = _rev(b1), _rev(b0)
  l0, l1 = jnp.minimum(a0, r0), jnp.minimum(a1, r1)
  h0, h1 = jnp.maximum(a0, r0), jnp.maximum(a1, r1)
  c0 = _vsort(jnp.minimum(l0, l1))
  c1 = _vsort(jnp.maximum(l0, l1))
  c2 = _vsort(jnp.minimum(h0, h1))
  if m == 4:
    return [c0, c1, c2]
  rb = _rev(s[4])
  p0, p1 = jnp.minimum(c0, c2), jnp.minimum(c1, rb)
  q0, q1 = jnp.maximum(c0, c2), jnp.maximum(c1, rb)
  return [_vsort(jnp.minimum(p0, p1)), _vsort(jnp.maximum(p0, p1)),
          _vsort(jnp.minimum(q0, q1))]


def _sc_body(hist_hbm, rat_hbm, cu_hbm, film_hbm, table_hbm,
             ue_hbm, fe_hbm,
             idx_a, rat_a, rows_a, idx_b, rat_b, rows_b,
             chmaj, sorted_v, cu_v, film_v, ferows_v, ue_v, sem):
  wid = lax.axis_index("s") * NC + lax.axis_index("c")
  u0 = wid * UPW

  pltpu.sync_copy(cu_hbm.at[pl.ds(u0, UPW + 16)], cu_v)
  pltpu.sync_copy(film_hbm.at[pl.ds(u0, UPW)], film_v)
  pltpu.async_copy(table_hbm.at[film_v], ferows_v, sem).wait()
  pltpu.sync_copy(ferows_v, fe_hbm.at[pl.ds(u0, UPW)])

  iota = lax.iota(_I32, L)
  inf16 = jnp.full((L,), jnp.inf, _F32)
  bufs = ((idx_a, rat_a, rows_a), (idx_b, rat_b, rows_b))

  def start_fetch(u, buf):
    bi, br, brw = buf
    cu_u = cu_v[pl.ds(u, L)][0]
    base = (cu_u // 8) * 8
    pltpu.sync_copy(hist_hbm.at[pl.ds(base, FETCH)], bi)
    pltpu.sync_copy(rat_hbm.at[pl.ds(base, FETCH)], br.at[0, pl.ds(0, FETCH)])
    pltpu.async_copy(table_hbm.at[bi], brw, sem)

  def process(u, cur, nxt):
    _, rat_v, rows_v = cur
    pltpu.make_async_copy(table_hbm.at[pl.ds(0, FETCH)], rows_v, sem).wait()

    @pl.when(u + 1 < UPW)
    def _prefetch():
      start_fetch(u + 1, nxt)

    cu_pair = cu_v[pl.ds(u, L)]
    cu_u = cu_pair[0]
    ln = cu_pair[1] - cu_u
    off = cu_u - (cu_u // 8) * 8

    chs = [iota + (g * L) for g in range(4)]
    zero16 = jnp.zeros((L,), _F32)
    ninf16 = jnp.full((L,), -jnp.inf, _F32)

    def t_main(t, carry):
      acc, mx = carry
      r = rat_v[0, pl.ds(off + t, L)][0]
      tvec = jnp.full((L,), t, _I32)
      accn, mxn = [], []
      for g in range(4):
        w = rows_v[off + t, pl.ds(g * L, L)] * r
        plsc.store_scatter(chmaj, [chs[g], tvec], w)
        accn.append(acc[g] + w)
        mxn.append(jnp.maximum(mx[g], w))
      return tuple(accn), tuple(mxn)

    acc, mx = lax.fori_loop(0, ln, t_main, ((zero16,) * 4, (ninf16,) * 4))

    m16 = ((ln + 15) // 16) * 16

    def t_pad(t, _):
      tvec = jnp.full((L,), t, _I32)
      for g in range(4):
        plsc.store_scatter(chmaj, [chs[g], tvec], inf16)
      return 0

    lax.fori_loop(ln, m16, t_pad, 0)

    def make_sort_phase(m):
      def chan_body(c, _):
        v = [chmaj[c, pl.ds(g * L, L)] for g in range(m)]
        srt = _sort_low(v)
        for j in range(len(srt)):
          sorted_v[c, pl.ds(j * L, L)] = srt[j]
        return 0

      return lambda: lax.fori_loop(0, E, chan_body, 0, unroll=2)

    lax.switch(jnp.clip((ln + 15) // 16 - 2, 0, 3),
               [make_sort_phase(m) for m in (2, 3, 4, 5)])

    ln_vec = jnp.full((L,), ln, _I32).astype(_F32)
    k_med = jnp.full((L,), (ln - 1) // 2, _I32)
    k_min = jnp.zeros((L,), _I32)
    for g in range(4):
      ue_v[u, pl.ds(g * L, L)] = plsc.load_gather(sorted_v, [chs[g], k_min])
      ue_v[u, pl.ds(E + g * L, L)] = mx[g]
      ue_v[u, pl.ds(2 * E + g * L, L)] = acc[g] / ln_vec
      ue_v[u, pl.ds(3 * E + g * L, L)] = plsc.load_gather(
          sorted_v, [chs[g], k_med])

  start_fetch(0, bufs[0])

  def pair_body(v, _):
    process(2 * v, bufs[0], bufs[1])
    process(2 * v + 1, bufs[1], bufs[0])
    return 0

  lax.fori_loop(0, UPW // 2, pair_body, 0)
  pltpu.sync_copy(ue_v, ue_hbm.at[pl.ds(u0, UPW)])


def _sc_stats(hist_idx, ratings, cu, film, table):
  mesh = plsc.VectorSubcoreMesh(core_axis_name="c", subcore_axis_name="s")
  fn = functools.partial(
      pl.kernel,
      mesh=mesh,
      compiler_params=pltpu.CompilerParams(
          needs_layout_passes=False, use_tc_tiling_on_sc=False),
      out_type=[
          jax.ShapeDtypeStruct((B, 4 * E), _F32),
          jax.ShapeDtypeStruct((B, E), _F32),
      ],
      scratch_types=[
          pltpu.VMEM((FETCH,), _I32),
          pltpu.VMEM((1, RATW), _F32),
          pltpu.VMEM((FETCH, E), _F32),
          pltpu.VMEM((FETCH,), _I32),
          pltpu.VMEM((1, RATW), _F32),
          pltpu.VMEM((FETCH, E), _F32),
          pltpu.VMEM((E, 81), _F32),
          pltpu.VMEM((E, 49), _F32),
          pltpu.VMEM((UPW + 16,), _I32),
          pltpu.VMEM((UPW,), _I32),
          pltpu.VMEM((UPW, E), _F32),
          pltpu.VMEM((UPW, 4 * E), _F32),
          pltpu.SemaphoreType.DMA,
      ],
  )(_sc_body)
  return fn(hist_idx, ratings, cu, film, table)


def _mlp_body(ue_ref, fe_ref, w1_ref, b1_ref, w2_ref, b2_ref, w3_ref, b3_ref,
              out_ref):
  ue = ue_ref[...]
  fe = fe_ref[...]
  ss = jnp.sum(ue * ue, axis=1, keepdims=True)
  uen = ue * lax.rsqrt(ss)
  x = jnp.concatenate([uen, fe], axis=1)
  h = jnp.dot(x, w1_ref[...], preferred_element_type=_F32,
              precision=lax.Precision.HIGHEST) + b1_ref[...][None, :]
  h = jnp.maximum(h, 0.0)
  h = jnp.dot(h, w2_ref[...], preferred_element_type=_F32,
              precision=lax.Precision.HIGHEST) + b2_ref[...][None, :]
  h = jnp.maximum(h, 0.0)
  o = jnp.sum(h * w3_ref[...][None, :], axis=1) + b3_ref[...]
  out_ref[...] = 1.0 / (1.0 + jnp.exp(-o))


def _mlp(ue, fe, W1, b1, W2, b2, W3, b3):
  UB = 512
  grid = (B // UB,)
  return pl.pallas_call(
      _mlp_body,
      grid=grid,
      in_specs=[
          pl.BlockSpec((UB, 4 * E), lambda i: (i, 0)),
          pl.BlockSpec((UB, E), lambda i: (i, 0)),
          pl.BlockSpec((5 * E, 2 * E), lambda i: (0, 0)),
          pl.BlockSpec((2 * E,), lambda i: (0,)),
          pl.BlockSpec((2 * E, E), lambda i: (0, 0)),
          pl.BlockSpec((E,), lambda i: (0,)),
          pl.BlockSpec((E,), lambda i: (0,)),
          pl.BlockSpec((1,), lambda i: (0,)),
      ],
      out_specs=pl.BlockSpec((UB,), lambda i: (i,)),
      out_shape=jax.ShapeDtypeStruct((B,), _F32),
  )(ue, fe, W1, b1, W2, b2, W3[:, 0], b3)


def kernel(hist_film_indices, hist_ratings, seg_ids, cu_seqlens, lengths,
           film_indices, table, W1, b1, W2, b2, W3, b3):
  del seg_ids, lengths
  total = hist_film_indices.shape[0]
  hist_p = jnp.concatenate(
      [hist_film_indices.astype(_I32), jnp.zeros((FETCH,), _I32)])
  rat_p = jnp.concatenate(
      [hist_ratings.astype(_F32), jnp.zeros((FETCH,), _F32)])
  cu_p = jnp.concatenate(
      [cu_seqlens.astype(_I32), jnp.full((16,), total, _I32)])
  ue, fe = _sc_stats(hist_p, rat_p, cu_p, film_indices.astype(_I32), table)
  return _mlp(ue, fe, W1, b1, W2, b2, W3, b3)


# speedup vs baseline: 1.3827x; 1.1856x over previous
import functools

import jax
import jax.numpy as jnp
from jax import lax
from jax.experimental import pallas as pl
from jax.experimental.pallas import tpu as pltpu
from jax.experimental.pallas import tpu_sc as plsc

B = 4096
E = 64
NC, NS, L = 2, 16, 16
NW = NC * NS
UPW = B // NW
LP = 80
NG = LP // L
FETCH = 88
RATW = 96

_F32 = jnp.float32
_I32 = jnp.int32


def _vsort(x):
  return jnp.sort(x)


def _rev(x):
  return lax.rev(x, (0,))


def _merge2(a, b):
  br = _rev(b)
  return _vsort(jnp.minimum(a, br)), _vsort(jnp.maximum(a, br))


def _sort_low(v):
  m = len(v)
  s = [_vsort(x) for x in v]
  if m == 2:
    return list(_merge2(s[0], s[1]))
  if m == 3:
    a0, a1 = _merge2(s[0], s[1])
    l1 = jnp.minimum(a1, _rev(s[2]))
    return [_vsort(jnp.minimum(a0, l1)), _vsort(jnp.maximum(a0, l1))]
  a0, a1 = _merge2(s[0], s[1])
  b0, b1 = _merge2(s[2], s[3])
  r0, r1 = _rev(b1), _rev(b0)
  l0, l1 = jnp.minimum(a0, r0), jnp.minimum(a1, r1)
  h0, h1 = jnp.maximum(a0, r0), jnp.maximum(a1, r1)
  c0 = _vsort(jnp.minimum(l0, l1))
  c1 = _vsort(jnp.maximum(l0, l1))
  c2 = _vsort(jnp.minimum(h0, h1))
  if m == 4:
    return [c0, c1, c2]
  rb = _rev(s[4])
  p0, p1 = jnp.minimum(c0, c2), jnp.minimum(c1, rb)
  q0, q1 = jnp.maximum(c0, c2), jnp.maximum(c1, rb)
  return [_vsort(jnp.minimum(p0, p1)), _vsort(jnp.maximum(p0, p1)),
          _vsort(jnp.minimum(q0, q1))]


def _sc_body(hist_hbm, rat_hbm, cu_hbm, film_hbm, table_hbm,
             ue_hbm, fe_hbm,
             idx_a, rat_a, rows_a, idx_b, rat_b, rows_b,
             chmaj, sorted_v, cu_v, film_v, ferows_v, ue_v, sem, sem2):
  wid = lax.axis_index("s") * NC + lax.axis_index("c")
  u0 = wid * UPW

  pltpu.sync_copy(cu_hbm.at[pl.ds(u0, UPW + 16)], cu_v)
  pltpu.sync_copy(film_hbm.at[pl.ds(u0, UPW)], film_v)
  pltpu.async_copy(table_hbm.at[film_v], ferows_v, sem).wait()
  pltpu.sync_copy(ferows_v, fe_hbm.at[pl.ds(u0, UPW)])

  iota = lax.iota(_I32, L)
  inf16 = jnp.full((L,), jnp.inf, _F32)
  bufs = ((idx_a, rat_a, rows_a), (idx_b, rat_b, rows_b))

  def issue_stage(u, buf):
    bi, br, _ = buf
    cu_u = cu_v[pl.ds(u, L)][0]
    base = (cu_u // 8) * 8
    pltpu.async_copy(hist_hbm.at[pl.ds(base, FETCH)], bi, sem2)
    pltpu.async_copy(rat_hbm.at[pl.ds(base, FETCH)],
                     br.at[0, pl.ds(0, FETCH)], sem2)

  def drain_stage(buf):
    bi, br, _ = buf
    pltpu.make_async_copy(hist_hbm.at[pl.ds(0, FETCH)], bi, sem2).wait()
    pltpu.make_async_copy(rat_hbm.at[pl.ds(0, FETCH)],
                          br.at[0, pl.ds(0, FETCH)], sem2).wait()

  def gather_stage(buf):
    bi, _, brw = buf
    pltpu.async_copy(table_hbm.at[bi], brw, sem)

  def process(u, cur, nxt):
    _, rat_v, rows_v = cur
    pltpu.make_async_copy(table_hbm.at[pl.ds(0, FETCH)], rows_v, sem).wait()

    @pl.when(u + 1 < UPW)
    def _prefetch_issue():
      issue_stage(u + 1, nxt)

    cu_pair = cu_v[pl.ds(u, L)]
    cu_u = cu_pair[0]
    ln = cu_pair[1] - cu_u
    off = cu_u - (cu_u // 8) * 8

    chs = [iota + (g * L) for g in range(4)]
    zero16 = jnp.zeros((L,), _F32)
    ninf16 = jnp.full((L,), -jnp.inf, _F32)

    def t_main(t, carry):
      acc, mx = carry
      r = rat_v[0, pl.ds(off + t, L)][0]
      tvec = jnp.full((L,), t, _I32)
      accn, mxn = [], []
      for g in range(4):
        w = rows_v[off + t, pl.ds(g * L, L)] * r
        plsc.store_scatter(chmaj, [chs[g], tvec], w)
        accn.append(acc[g] + w)
        mxn.append(jnp.maximum(mx[g], w))
      return tuple(accn), tuple(mxn)

    acc, mx = lax.fori_loop(0, ln, t_main, ((zero16,) * 4, (ninf16,) * 4))

    m16 = ((ln + 15) // 16) * 16

    def t_pad(t, _):
      tvec = jnp.full((L,), t, _I32)
      for g in range(4):
        plsc.store_scatter(chmaj, [chs[g], tvec], inf16)
      return 0

    lax.fori_loop(ln, m16, t_pad, 0)

    @pl.when(u + 1 < UPW)
    def _prefetch_gather():
      drain_stage(nxt)
      gather_stage(nxt)

    def make_sort_phase(m):
      def chan_body(c, _):
        v = [chmaj[c, pl.ds(g * L, L)] for g in range(m)]
        srt = _sort_low(v)
        for j in range(len(srt)):
          sorted_v[c, pl.ds(j * L, L)] = srt[j]
        return 0

      return lambda: lax.fori_loop(0, E, chan_body, 0, unroll=2)

    lax.switch(jnp.clip((ln + 15) // 16 - 2, 0, 3),
               [make_sort_phase(m) for m in (2, 3, 4, 5)])

    ln_vec = jnp.full((L,), ln, _I32).astype(_F32)
    k_med = jnp.full((L,), (ln - 1) // 2, _I32)
    k_min = jnp.zeros((L,), _I32)
    for g in range(4):
      ue_v[u, pl.ds(g * L, L)] = plsc.load_gather(sorted_v, [chs[g], k_min])
      ue_v[u, pl.ds(E + g * L, L)] = mx[g]
      ue_v[u, pl.ds(2 * E + g * L, L)] = acc[g] / ln_vec
      ue_v[u, pl.ds(3 * E + g * L, L)] = plsc.load_gather(
          sorted_v, [chs[g], k_med])

  issue_stage(0, bufs[0])
  drain_stage(bufs[0])
  gather_stage(bufs[0])

  def pair_body(v, _):
    process(2 * v, bufs[0], bufs[1])
    process(2 * v + 1, bufs[1], bufs[0])
    return 0

  lax.fori_loop(0, UPW // 2, pair_body, 0)
  pltpu.sync_copy(ue_v, ue_hbm.at[pl.ds(u0, UPW)])


def _sc_stats(hist_idx, ratings, cu, film, table):
  mesh = plsc.VectorSubcoreMesh(core_axis_name="c", subcore_axis_name="s")
  fn = functools.partial(
      pl.kernel,
      mesh=mesh,
      compiler_params=pltpu.CompilerParams(
          needs_layout_passes=False, use_tc_tiling_on_sc=False),
      out_type=[
          jax.ShapeDtypeStruct((B, 4 * E), _F32),
          jax.ShapeDtypeStruct((B, E), _F32),
      ],
      scratch_types=[
          pltpu.VMEM((FETCH,), _I32),
          pltpu.VMEM((1, RATW), _F32),
          pltpu.VMEM((FETCH, E), _F32),
          pltpu.VMEM((FETCH,), _I32),
          pltpu.VMEM((1, RATW), _F32),
          pltpu.VMEM((FETCH, E), _F32),
          pltpu.VMEM((E, 81), _F32),
          pltpu.VMEM((E, 49), _F32),
          pltpu.VMEM((UPW + 16,), _I32),
          pltpu.VMEM((UPW,), _I32),
          pltpu.VMEM((UPW, E), _F32),
          pltpu.VMEM((UPW, 4 * E), _F32),
          pltpu.SemaphoreType.DMA,
          pltpu.SemaphoreType.DMA,
      ],
  )(_sc_body)
  return fn(hist_idx, ratings, cu, film, table)


def _mlp_body(ue_ref, fe_ref, w1_ref, b1_ref, w2_ref, b2_ref, w3_ref, b3_ref,
              out_ref):
  ue = ue_ref[...]
  fe = fe_ref[...]
  ss = jnp.sum(ue * ue, axis=1, keepdims=True)
  uen = ue * lax.rsqrt(ss)
  x = jnp.concatenate([uen, fe], axis=1)
  h = jnp.dot(x, w1_ref[...], preferred_element_type=_F32,
              precision=lax.Precision.HIGHEST) + b1_ref[...][None, :]
  h = jnp.maximum(h, 0.0)
  h = jnp.dot(h, w2_ref[...], preferred_element_type=_F32,
              precision=lax.Precision.HIGHEST) + b2_ref[...][None, :]
  h = jnp.maximum(h, 0.0)
  o = jnp.sum(h * w3_ref[...][None, :], axis=1) + b3_ref[...]
  out_ref[...] = 1.0 / (1.0 + jnp.exp(-o))


def _mlp(ue, fe, W1, b1, W2, b2, W3, b3):
  UB = 512
  grid = (B // UB,)
  return pl.pallas_call(
      _mlp_body,
      grid=grid,
      in_specs=[
          pl.BlockSpec((UB, 4 * E), lambda i: (i, 0)),
          pl.BlockSpec((UB, E), lambda i: (i, 0)),
          pl.BlockSpec((5 * E, 2 * E), lambda i: (0, 0)),
          pl.BlockSpec((2 * E,), lambda i: (0,)),
          pl.BlockSpec((2 * E, E), lambda i: (0, 0)),
          pl.BlockSpec((E,), lambda i: (0,)),
          pl.BlockSpec((E,), lambda i: (0,)),
          pl.BlockSpec((1,), lambda i: (0,)),
      ],
      out_specs=pl.BlockSpec((UB,), lambda i: (i,)),
      out_shape=jax.ShapeDtypeStruct((B,), _F32),
  )(ue, fe, W1, b1, W2, b2, W3[:, 0], b3)


def kernel(hist_film_indices, hist_ratings, seg_ids, cu_seqlens, lengths,
           film_indices, table, W1, b1, W2, b2, W3, b3):
  del seg_ids, lengths
  total = hist_film_indices.shape[0]
  hist_p = jnp.concatenate(
      [hist_film_indices.astype(_I32), jnp.zeros((FETCH,), _I32)])
  rat_p = jnp.concatenate(
      [hist_ratings.astype(_F32), jnp.zeros((FETCH,), _F32)])
  cu_p = jnp.concatenate(
      [cu_seqlens.astype(_I32), jnp.full((16,), total, _I32)])
  ue, fe = _sc_stats(hist_p, rat_p, cu_p, film_indices.astype(_I32), table)
  return _mlp(ue, fe, W1, b1, W2, b2, W3, b3)
